# Initial kernel scaffold; baseline (speedup 1.0000x reference)
import jax
import jax.numpy as jnp
from jax.experimental import pallas as pl


def kernel(x):
    raise NotImplementedError("write your pallas kernel here")


# speedup vs baseline: 1.1059x; 1.1059x over previous
import functools

import numpy as np
import jax
import jax.numpy as jnp
from jax import lax
from jax.experimental import pallas as pl
from jax.experimental.pallas import tpu as pltpu
from jax.experimental.pallas import tpu_sc as plsc

_H = 224
_PAD = 243
_NSEQ = _H * _H
_NC, _NS, _L = 2, 16, 16
_NW = _NC * _NS
_NIMG = 8 * 96
_IMGS_PW = _NIMG // _NW
_NCHUNK = 8
_CHUNK = _NSEQ // _NCHUNK


def _peano_coords(level):
    if level == 0:
        return [(0, 0)]
    sub = _peano_coords(level - 1)
    size = 3 ** (level - 1)
    blocks = [(0, 0, 0), (0, 1, 0), (0, 2, 0), (1, 2, 1), (1, 1, 1),
              (1, 0, 1), (2, 0, 0), (2, 1, 0), (2, 2, 0)]
    out = []
    for bx, by, rot in blocks:
        for x, y in sub:
            if rot:
                x, y = (y, x)
            out.append((bx * size + x, by * size + y))
    return out


def _source_indices() -> np.ndarray:
    coords = _peano_coords(5)[:_NSEQ]
    rr = np.array([r for r, _ in coords])
    cc = np.array([c for _, c in coords])
    src = np.full(_NSEQ, _NSEQ, dtype=np.int32)
    valid = (rr < _H) & (cc < _H)
    src[valid] = (rr[valid] * _H + cc[valid]).astype(np.int32)
    return src


_SRC = _source_indices()


def _sc_body(x_hbm, idx_hbm, out_hbm, idx_v, img_v, out_v):
    wid = lax.axis_index("s") * _NC + lax.axis_index("c")
    pltpu.sync_copy(idx_hbm, idx_v)
    img_v[pl.ds(_NSEQ, _L)] = jnp.zeros((_L,), jnp.float32)

    def per_image(n, _):
        row = wid * _IMGS_PW + n
        base = row * _NSEQ
        pltpu.sync_copy(x_hbm.at[pl.ds(base, _NSEQ)], img_v.at[pl.ds(0, _NSEQ)])

        def per_chunk(k, _):
            def per_vec(j, _):
                iv = idx_v[pl.ds(k * _CHUNK + j * _L, _L)]
                out_v[pl.ds(j * _L, _L)] = plsc.load_gather(img_v, [iv])
                return 0

            lax.fori_loop(0, _CHUNK // _L, per_vec, 0, unroll=8)
            pltpu.sync_copy(out_v, out_hbm.at[pl.ds(base + k * _CHUNK, _CHUNK)])
            return 0

        lax.fori_loop(0, _NCHUNK, per_chunk, 0)
        return 0

    lax.fori_loop(0, _IMGS_PW, per_image, 0)


@functools.partial(jax.jit, static_argnums=())
def _peano_gather(xf, src):
    mesh = plsc.VectorSubcoreMesh(core_axis_name="c", subcore_axis_name="s")
    f = pl.kernel(
        _sc_body,
        out_type=jax.ShapeDtypeStruct((_NIMG * _NSEQ,), jnp.float32),
        mesh=mesh,
        scratch_types=[
            pltpu.VMEM((_NSEQ,), jnp.int32),
            pltpu.VMEM((_NSEQ + _L,), jnp.float32),
            pltpu.VMEM((_CHUNK,), jnp.float32),
        ],
        compiler_params=pltpu.CompilerParams(needs_layout_passes=False),
    )
    return f(xf, src)


def kernel(x):
    B, C, H, W = x.shape
    assert (B * C, H, W) == (_NIMG, _H, _H)
    xf = x.reshape(B * C * H * W)
    out = _peano_gather(xf, jnp.asarray(_SRC))
    return out.reshape(B, C, _NSEQ)


# speedup vs baseline: 2.3400x; 2.1160x over previous
import functools

import numpy as np
import jax
import jax.numpy as jnp
from jax import lax
from jax.experimental import pallas as pl
from jax.experimental.pallas import tpu as pltpu
from jax.experimental.pallas import tpu_sc as plsc

_H = 224
_PAD = 243
_NSEQ = _H * _H
_NC, _NS, _L = 2, 16, 16
_NW = _NC * _NS
_NIMG = 8 * 96
_IMGS_PW = _NIMG // _NW
_NCHUNK = 8
_CHUNK = _NSEQ // _NCHUNK


def _peano_coords(level):
    if level == 0:
        return [(0, 0)]
    sub = _peano_coords(level - 1)
    size = 3 ** (level - 1)
    blocks = [(0, 0, 0), (0, 1, 0), (0, 2, 0), (1, 2, 1), (1, 1, 1),
              (1, 0, 1), (2, 0, 0), (2, 1, 0), (2, 2, 0)]
    out = []
    for bx, by, rot in blocks:
        for x, y in sub:
            if rot:
                x, y = (y, x)
            out.append((bx * size + x, by * size + y))
    return out


def _source_indices() -> np.ndarray:
    coords = _peano_coords(5)[:_NSEQ]
    rr = np.array([r for r, _ in coords])
    cc = np.array([c for _, c in coords])
    src = np.full(_NSEQ, _NSEQ, dtype=np.int32)
    valid = (rr < _H) & (cc < _H)
    src[valid] = (rr[valid] * _H + cc[valid]).astype(np.int32)
    return src


_SRC = _source_indices()


def _sc_body(x_hbm, idx_hbm, out_hbm, idx_v, img_v, out0, out1, sem0, sem1):
    wid = lax.axis_index("s") * _NC + lax.axis_index("c")
    pltpu.sync_copy(idx_hbm, idx_v)
    img_v[pl.ds(_NSEQ, _L)] = jnp.zeros((_L,), jnp.float32)

    bufs = (out0, out1)
    sems = (sem0, sem1)
    for b in range(2):
        pltpu.async_copy(out_hbm.at[pl.ds(b * _CHUNK, _CHUNK)], bufs[b], sems[b])

    def per_image(n, _):
        base = (wid * _IMGS_PW + n) * _NSEQ
        pltpu.sync_copy(x_hbm.at[pl.ds(base, _NSEQ)], img_v.at[pl.ds(0, _NSEQ)])
        for k in range(_NCHUNK):
            buf, sem = bufs[k % 2], sems[k % 2]
            pltpu.make_async_copy(buf, out_hbm.at[pl.ds(k * _CHUNK, _CHUNK)],
                                  sem).wait()

            @plsc.parallel_loop(0, _CHUNK, step=_L, unroll=8)
            def gather_vec(off):
                iv = idx_v[pl.ds(k * _CHUNK + off, _L)]
                buf[pl.ds(off, _L)] = plsc.load_gather(img_v, [iv])

            pltpu.async_copy(buf, out_hbm.at[pl.ds(base + k * _CHUNK, _CHUNK)],
                             sem)
        return 0

    lax.fori_loop(0, _IMGS_PW, per_image, 0)
    for b in range(2):
        pltpu.make_async_copy(bufs[b], out_hbm.at[pl.ds(b * _CHUNK, _CHUNK)],
                              sems[b]).wait()


@functools.partial(jax.jit, static_argnums=())
def _peano_gather(xf, src):
    mesh = plsc.VectorSubcoreMesh(core_axis_name="c", subcore_axis_name="s")
    f = pl.kernel(
        _sc_body,
        out_type=jax.ShapeDtypeStruct((_NIMG * _NSEQ,), jnp.float32),
        mesh=mesh,
        scratch_types=[
            pltpu.VMEM((_NSEQ,), jnp.int32),
            pltpu.VMEM((_NSEQ + _L,), jnp.float32),
            pltpu.VMEM((_CHUNK,), jnp.float32),
            pltpu.VMEM((_CHUNK,), jnp.float32),
            pltpu.SemaphoreType.DMA,
            pltpu.SemaphoreType.DMA,
        ],
        compiler_params=pltpu.CompilerParams(needs_layout_passes=False),
    )
    return f(xf, src)


def kernel(x):
    B, C, H, W = x.shape
    assert (B * C, H, W) == (_NIMG, _H, _H)
    xf = x.reshape(B * C * H * W)
    out = _peano_gather(xf, jnp.asarray(_SRC))
    return out.reshape(B, C, _NSEQ)


# speedup vs baseline: 2.9938x; 1.2794x over previous
import functools

import numpy as np
import jax
import jax.numpy as jnp
from jax import lax
from jax.experimental import pallas as pl
from jax.experimental.pallas import tpu as pltpu
from jax.experimental.pallas import tpu_sc as plsc

_H = 224
_PAD = 243
_NSEQ = _H * _H
_NC, _NS, _L = 2, 16, 16
_NW = _NC * _NS
_NIMG = 8 * 96
_IMGS_PW = _NIMG // _NW
_NCHUNK = 8
_CHUNK = _NSEQ // _NCHUNK


def _peano_coords(level):
    if level == 0:
        return [(0, 0)]
    sub = _peano_coords(level - 1)
    size = 3 ** (level - 1)
    blocks = [(0, 0, 0), (0, 1, 0), (0, 2, 0), (1, 2, 1), (1, 1, 1),
              (1, 0, 1), (2, 0, 0), (2, 1, 0), (2, 2, 0)]
    out = []
    for bx, by, rot in blocks:
        for x, y in sub:
            if rot:
                x, y = (y, x)
            out.append((bx * size + x, by * size + y))
    return out


def _source_indices() -> np.ndarray:
    coords = _peano_coords(5)[:_NSEQ]
    rr = np.array([r for r, _ in coords])
    cc = np.array([c for _, c in coords])
    src = np.full(_NSEQ, _NSEQ, dtype=np.int32)
    valid = (rr < _H) & (cc < _H)
    src[valid] = (rr[valid] * _H + cc[valid]).astype(np.int32)
    return src


_SRC = _source_indices()


def _sc_body(x_hbm, idx_hbm, out_hbm, idx_v, img_v, out0, out1, sem0, sem1):
    wid = lax.axis_index("s") * _NC + lax.axis_index("c")
    pltpu.sync_copy(idx_hbm, idx_v)
    img_v[pl.ds(_NSEQ, _L)] = jnp.zeros((_L,), jnp.float32)

    bufs = (out0, out1)
    sems = (sem0, sem1)
    for b in range(2):
        pltpu.async_copy(out_hbm.at[0, 0, pl.ds(b * _CHUNK, _CHUNK)],
                         bufs[b], sems[b])

    def per_image(n, _):
        row = wid * _IMGS_PW + n
        bi = row // 96
        ci = row - bi * 96
        base = row * _NSEQ
        pltpu.sync_copy(x_hbm.at[pl.ds(base, _NSEQ)], img_v.at[pl.ds(0, _NSEQ)])
        for k in range(_NCHUNK):
            buf, sem = bufs[k % 2], sems[k % 2]
            pltpu.make_async_copy(
                buf, out_hbm.at[0, 0, pl.ds(k * _CHUNK, _CHUNK)], sem).wait()

            @plsc.parallel_loop(0, _CHUNK, step=_L, unroll=8)
            def gather_vec(off):
                iv = idx_v[pl.ds(k * _CHUNK + off, _L)]
                buf[pl.ds(off, _L)] = plsc.load_gather(img_v, [iv])

            pltpu.async_copy(buf,
                             out_hbm.at[bi, ci, pl.ds(k * _CHUNK, _CHUNK)],
                             sem)
        return 0

    lax.fori_loop(0, _IMGS_PW, per_image, 0)
    for b in range(2):
        pltpu.make_async_copy(bufs[b], out_hbm.at[0, 0, pl.ds(b * _CHUNK, _CHUNK)],
                              sems[b]).wait()


@functools.partial(jax.jit, static_argnums=())
def _peano_gather(xf, src):
    mesh = plsc.VectorSubcoreMesh(core_axis_name="c", subcore_axis_name="s")
    f = pl.kernel(
        _sc_body,
        out_type=jax.ShapeDtypeStruct((8, 96, _NSEQ), jnp.float32),
        mesh=mesh,
        scratch_types=[
            pltpu.VMEM((_NSEQ,), jnp.int32),
            pltpu.VMEM((_NSEQ + _L,), jnp.float32),
            pltpu.VMEM((_CHUNK,), jnp.float32),
            pltpu.VMEM((_CHUNK,), jnp.float32),
            pltpu.SemaphoreType.DMA,
            pltpu.SemaphoreType.DMA,
        ],
        compiler_params=pltpu.CompilerParams(needs_layout_passes=False),
    )
    return f(xf, src)


def kernel(x):
    B, C, H, W = x.shape
    assert (B * C, H, W) == (_NIMG, _H, _H)
    xf = x.reshape(B * C * H * W)
    return _peano_gather(xf, jnp.asarray(_SRC))


# speedup vs baseline: 5.1947x; 1.7351x over previous
import functools

import numpy as np
import jax
import jax.numpy as jnp
from jax import lax
from jax.experimental import pallas as pl
from jax.experimental.pallas import tpu as pltpu
from jax.experimental.pallas import tpu_sc as plsc

_H = 224
_PAD = 243
_NSEQ = _H * _H
_NC, _NS, _L = 2, 16, 16
_NW = _NC * _NS
_NIMG = 8 * 96
_IMGS_PW = _NIMG // _NW
_NCHUNK = 8
_CHUNK = _NSEQ // _NCHUNK


def _peano_coords(level):
    if level == 0:
        return [(0, 0)]
    sub = _peano_coords(level - 1)
    size = 3 ** (level - 1)
    blocks = [(0, 0, 0), (0, 1, 0), (0, 2, 0), (1, 2, 1), (1, 1, 1),
              (1, 0, 1), (2, 0, 0), (2, 1, 0), (2, 2, 0)]
    out = []
    for bx, by, rot in blocks:
        for x, y in sub:
            if rot:
                x, y = (y, x)
            out.append((bx * size + x, by * size + y))
    return out


def _source_indices() -> np.ndarray:
    coords = _peano_coords(5)[:_NSEQ]
    rr = np.array([r for r, _ in coords])
    cc = np.array([c for _, c in coords])
    src = np.full(_NSEQ, _H << 8, dtype=np.int32)
    valid = (rr < _H) & (cc < _H)
    src[valid] = ((rr[valid] << 8) | cc[valid]).astype(np.int32)
    return src


_SRC = _source_indices()


def _sc_body(x_hbm, idx_hbm, out_hbm, idx_v, img_v, out0, out1, sem0, sem1):
    wid = lax.axis_index("s") * _NC + lax.axis_index("c")
    pltpu.sync_copy(idx_hbm, idx_v)
    img_v[_H, pl.ds(0, _L)] = jnp.zeros((_L,), jnp.float32)

    bufs = (out0, out1)
    sems = (sem0, sem1)
    for b in range(2):
        pltpu.async_copy(out_hbm.at[0, 0, pl.ds(b * _CHUNK, _CHUNK)],
                         bufs[b], sems[b])

    def per_image(n, _):
        row = wid * _IMGS_PW + n
        bi = row // 96
        ci = row - bi * 96
        pltpu.sync_copy(x_hbm.at[bi, ci], img_v.at[pl.ds(0, _H), :])
        for k in range(_NCHUNK):
            buf, sem = bufs[k % 2], sems[k % 2]
            pltpu.make_async_copy(
                buf, out_hbm.at[0, 0, pl.ds(k * _CHUNK, _CHUNK)], sem).wait()

            @plsc.parallel_loop(0, _CHUNK, step=_L, unroll=8)
            def gather_vec(off):
                iv = idx_v[pl.ds(k * _CHUNK + off, _L)]
                ivr = lax.shift_right_logical(iv, 8)
                ivc = lax.bitwise_and(iv, 255)
                buf[pl.ds(off, _L)] = plsc.load_gather(img_v, [ivr, ivc])

            pltpu.async_copy(buf,
                             out_hbm.at[bi, ci, pl.ds(k * _CHUNK, _CHUNK)],
                             sem)
        return 0

    lax.fori_loop(0, _IMGS_PW, per_image, 0)
    for b in range(2):
        pltpu.make_async_copy(bufs[b], out_hbm.at[0, 0, pl.ds(b * _CHUNK, _CHUNK)],
                              sems[b]).wait()


@functools.partial(jax.jit, static_argnums=())
def _peano_gather(xf, src):
    mesh = plsc.VectorSubcoreMesh(core_axis_name="c", subcore_axis_name="s")
    f = pl.kernel(
        _sc_body,
        out_type=jax.ShapeDtypeStruct((8, 96, _NSEQ), jnp.float32),
        name="peano_sc_gather",
        mesh=mesh,
        scratch_types=[
            pltpu.VMEM((_NSEQ,), jnp.int32),
            pltpu.VMEM((_H + 8, _H), jnp.float32),
            pltpu.VMEM((_CHUNK,), jnp.float32),
            pltpu.VMEM((_CHUNK,), jnp.float32),
            pltpu.SemaphoreType.DMA,
            pltpu.SemaphoreType.DMA,
        ],
        compiler_params=pltpu.CompilerParams(needs_layout_passes=False),
    )
    return f(xf, src)


def kernel(x):
    B, C, H, W = x.shape
    assert (B * C, H, W) == (_NIMG, _H, _H)
    return _peano_gather(x, jnp.asarray(_SRC))
